# Initial kernel scaffold; baseline (speedup 1.0000x reference)
import jax
import jax.numpy as jnp
from jax.experimental import pallas as pl


def kernel(depth_ids, subtree_ids, depth_table, subtree_table):
    raise NotImplementedError("write your pallas kernel here")


# speedup vs baseline: 6.4311x; 6.4311x over previous
import jax
import jax.numpy as jnp
from jax import lax
from jax.experimental import pallas as pl

_ROWS_PER_BLOCK = 2048
_TBL = 128


def _embed_block(d_ref, s_ref, t_ref, o_ref):
    r = d_ref.shape[0]
    d = d_ref[...]
    s = s_ref[...]
    col = lax.broadcasted_iota(jnp.int32, (r, _TBL), 1)
    onehot = (col == d).astype(jnp.bfloat16) + (col == (s + 20)).astype(jnp.bfloat16)
    t = t_ref[...]
    t_hi = t.astype(jnp.bfloat16)
    t_lo = (t - t_hi.astype(jnp.float32)).astype(jnp.bfloat16)
    acc = jnp.dot(onehot, t_hi, preferred_element_type=jnp.float32)
    acc = acc + jnp.dot(onehot, t_lo, preferred_element_type=jnp.float32)
    o_ref[...] = acc


def kernel(depth_ids, subtree_ids, depth_table, subtree_table):
    b, sq = depth_ids.shape
    d_model = depth_table.shape[1]
    n = b * sq
    r = _ROWS_PER_BLOCK

    d_col = depth_ids.reshape(n, 1).astype(jnp.int32)
    s_col = subtree_ids.reshape(n, 1).astype(jnp.int32)
    tcat = jnp.zeros((_TBL, d_model), jnp.float32)
    tcat = tcat.at[: depth_table.shape[0]].set(depth_table)
    tcat = tcat.at[20 : 20 + subtree_table.shape[0]].set(subtree_table)

    out = pl.pallas_call(
        _embed_block,
        grid=(n // r,),
        in_specs=[
            pl.BlockSpec((r, 1), lambda i: (i, 0)),
            pl.BlockSpec((r, 1), lambda i: (i, 0)),
            pl.BlockSpec((_TBL, d_model), lambda i: (0, 0)),
        ],
        out_specs=pl.BlockSpec((r, d_model), lambda i: (i, 0)),
        out_shape=jax.ShapeDtypeStruct((n, d_model), jnp.float32),
    )(d_col, s_col, tcat)
    return out.reshape(b, sq, d_model)


# speedup vs baseline: 11.7864x; 1.8327x over previous
import functools

import jax
import jax.numpy as jnp
from jax.experimental import pallas as pl
from jax.experimental.pallas import tpu as pltpu
from jax.experimental.pallas import tpu_sc as plsc

_D = 128
_WINDOW = 128
_NTBL = 1024


def _prep_body(d_ids_ref, s_ids_ref, dt_ref, st_ref, t3_ref, cidx_ref):
    cidx_ref[...] = d_ids_ref[...] * 50 + s_ids_ref[...]
    dt = dt_ref[...]
    st = st_ref[...]
    t3_ref[...] = dt[:, None, :] + st[None, :, :]


def _sc_gather(table, cidx, n):
    mesh = plsc.VectorSubcoreMesh(core_axis_name="c", subcore_axis_name="s")

    @functools.partial(
        pl.kernel,
        out_type=jax.ShapeDtypeStruct((n, _D), jnp.float32),
        mesh=mesh,
    )
    def k(tbl_hbm, idx_hbm, out_hbm):
        def body(i_vmem, o_vmem):
            pltpu.sync_copy(tbl_hbm.at[i_vmem.at[0]], o_vmem)

        pltpu.emit_pipeline(
            body,
            grid=(n // _WINDOW,),
            in_specs=[pl.BlockSpec((1, _WINDOW), lambda i: (0, i))],
            out_specs=[pl.BlockSpec((_WINDOW, _D), lambda i: (i, 0))],
            core_axis_name=("c", "s"),
            dimension_semantics=(pltpu.PARALLEL,),
        )(idx_hbm, out_hbm)

    return k(table, cidx)


def kernel(depth_ids, subtree_ids, depth_table, subtree_table):
    b, sq = depth_ids.shape
    nd, d_model = depth_table.shape
    ns = subtree_table.shape[0]
    n = b * sq

    d_ids2 = depth_ids.reshape(n // 128, 128).astype(jnp.int32)
    s_ids2 = subtree_ids.reshape(n // 128, 128).astype(jnp.int32)

    t3, cidx2 = pl.pallas_call(
        _prep_body,
        out_shape=[
            jax.ShapeDtypeStruct((nd, ns, d_model), jnp.float32),
            jax.ShapeDtypeStruct((n // 128, 128), jnp.int32),
        ],
    )(d_ids2, s_ids2, depth_table, subtree_table)

    table = t3.reshape(nd * ns, d_model)
    cidx = cidx2.reshape(1, n)

    out = _sc_gather(table, cidx, n)
    return out.reshape(b, sq, d_model)


# speedup vs baseline: 28.4883x; 2.4170x over previous
import functools

import jax
import jax.numpy as jnp
from jax import lax
from jax.experimental import pallas as pl
from jax.experimental.pallas import tpu as pltpu
from jax.experimental.pallas import tpu_sc as plsc

_D = 128
_WINDOW = 128
_NTBL = 1024


def _prep_body(d_ids_ref, s_ids_ref, dt_ref, st_ref, t3_ref, cidx_ref):
    cidx_ref[...] = d_ids_ref[...] * 50 + s_ids_ref[...]
    dt = dt_ref[...]
    st = st_ref[...]
    t3_ref[...] = dt[:, None, :] + st[None, :, :]


def _sc_gather(table, cidx, n):
    mesh = plsc.VectorSubcoreMesh(core_axis_name="c", subcore_axis_name="s")

    @functools.partial(
        pl.kernel,
        out_type=jax.ShapeDtypeStruct((n, _D), jnp.float32),
        mesh=mesh,
        scratch_types=[pltpu.VMEM_SHARED((1000, _D), jnp.float32)],
    )
    def k(tbl_hbm, idx_hbm, out_hbm, tbl_sh):
        @pl.when(lax.axis_index("s") == 0)
        def _():
            pltpu.sync_copy(tbl_hbm, tbl_sh)

        plsc.subcore_barrier()

        def body(i_vmem, o_vmem):
            pltpu.sync_copy(tbl_sh.at[i_vmem.at[0]], o_vmem)

        pltpu.emit_pipeline(
            body,
            grid=(n // _WINDOW,),
            in_specs=[pl.BlockSpec((1, _WINDOW), lambda i: (0, i))],
            out_specs=[pl.BlockSpec((_WINDOW, _D), lambda i: (i, 0))],
            core_axis_name=("c", "s"),
            dimension_semantics=(pltpu.PARALLEL,),
        )(idx_hbm, out_hbm)

    return k(table, cidx)


def kernel(depth_ids, subtree_ids, depth_table, subtree_table):
    b, sq = depth_ids.shape
    nd, d_model = depth_table.shape
    ns = subtree_table.shape[0]
    n = b * sq

    d_ids2 = depth_ids.reshape(n // 128, 128).astype(jnp.int32)
    s_ids2 = subtree_ids.reshape(n // 128, 128).astype(jnp.int32)

    t3, cidx2 = pl.pallas_call(
        _prep_body,
        out_shape=[
            jax.ShapeDtypeStruct((nd, ns, d_model), jnp.float32),
            jax.ShapeDtypeStruct((n // 128, 128), jnp.int32),
        ],
    )(d_ids2, s_ids2, depth_table, subtree_table)

    table = t3.reshape(nd * ns, d_model)
    cidx = cidx2.reshape(1, n)

    out = _sc_gather(table, cidx, n)
    return out.reshape(b, sq, d_model)
